# Initial kernel scaffold; baseline (speedup 1.0000x reference)
import jax
import jax.numpy as jnp
from jax.experimental import pallas as pl


def kernel(x, emb_table, W_ih, W_hh, b_ih, b_hh, W_fc, b_fc):
    raise NotImplementedError("write your pallas kernel here")


# speedup vs baseline: 6.5881x; 6.5881x over previous
import jax
import jax.numpy as jnp
from jax import lax
from jax.experimental import pallas as pl
from jax.experimental.pallas import tpu as pltpu

SEQ = 8192
EMB = 10
HID = 26
VOCAB = 26
NCLS = 26


def _fused_kernel(x_ref, emb_ref, wih_t_ref, whh_t_ref, wfc_t_ref,
                  bin_ref, bfc_ref, out_ref, a_ref, h_ref):
    t2 = jnp.dot(emb_ref[...], wih_t_ref[...],
                 preferred_element_type=jnp.float32) + bin_ref[...]
    idx = x_ref[...]
    oh = (idx == lax.broadcasted_iota(jnp.int32, (SEQ, VOCAB), 1)
          ).astype(jnp.float32)
    a_ref[...] = jnp.dot(oh, t2, preferred_element_type=jnp.float32)

    whh_t = whh_t_ref[...]

    def body(t, h):
        a = a_ref[pl.ds(t, 1), :]
        hn = jnp.tanh(a + jnp.dot(h, whh_t,
                                  preferred_element_type=jnp.float32))
        h_ref[pl.ds(t, 1), :] = hn
        return hn

    lax.fori_loop(0, SEQ, body, jnp.zeros((1, HID), jnp.float32), unroll=8)

    out_ref[...] = jnp.dot(h_ref[...], wfc_t_ref[...],
                           preferred_element_type=jnp.float32) + bfc_ref[...]


def kernel(x, emb_table, W_ih, W_hh, b_ih, b_hh, W_fc, b_fc):
    xr = x.reshape(SEQ, 1).astype(jnp.int32)
    bin_ = (b_ih + b_hh).reshape(1, HID)
    out = pl.pallas_call(
        _fused_kernel,
        out_shape=jax.ShapeDtypeStruct((SEQ, NCLS), jnp.float32),
        scratch_shapes=[pltpu.VMEM((SEQ, HID), jnp.float32),
                        pltpu.VMEM((SEQ, HID), jnp.float32)],
    )(xr, emb_table, W_ih.T, W_hh.T, W_fc.T, bin_,
      b_fc.reshape(1, NCLS))
    return out.reshape(1, SEQ, NCLS)
